# baseline (speedup 1.0000x reference)
import functools

import jax
import jax.numpy as jnp
import numpy as np
from jax import lax
from jax.experimental import pallas as pl
from jax.experimental.pallas import tpu as pltpu
from jax.experimental.pallas import tpu_sc as plsc

SEQ = 200
D = 64
BATCH = 4096
LANES = 16
NC, NS = 2, 16
NW = NC * NS

G = 2
CH = G * 128
BT = BATCH // 128
UNITS = SEQ * (BT // G)
PER_W = UNITS // NW
NBUF = 2
NJ = D // LANES

VOCAB = 1000000
TW = 512
HGRID = 977
HALF = HGRID * TW


def _tc_linearize():
    def body(x1_ref, x2_ref, o_ref):
        a = jnp.transpose(x1_ref[...])
        b = jnp.transpose(x2_ref[...])
        o_ref[...] = jnp.concatenate([a, b], axis=1)

    return pl.pallas_call(
        body,
        grid=(HGRID,),
        in_specs=[
            pl.BlockSpec((D, TW), lambda i: (0, i)),
            pl.BlockSpec((D, TW), lambda i: (0, i + HGRID)),
        ],
        out_specs=pl.BlockSpec((TW, 2 * D), lambda i: (i, 0)),
        out_shape=jax.ShapeDtypeStruct((HALF, 2 * D), jnp.float32),
    )


def _build():
    mesh = plsc.VectorSubcoreMesh(core_axis_name="c", subcore_axis_name="s")

    @functools.partial(
        pl.kernel,
        out_type=jax.ShapeDtypeStruct((SEQ, D // 8 * BT * 8, 128), jnp.float32),
        mesh=mesh,
        scratch_types=[
            [pltpu.VMEM((CH + LANES,), jnp.int32) for _ in range(NBUF)],
            [pltpu.VMEM((CH,), jnp.int32) for _ in range(NBUF)],
            [pltpu.VMEM((CH, 2 * D), jnp.float32) for _ in range(NBUF)],
            [pltpu.VMEM((G * D, 129), jnp.float32) for _ in range(NBUF)],
            pltpu.VMEM((SEQ, D), jnp.float32),
            pltpu.VMEM((NJ * G, LANES), jnp.int32),
            [pltpu.SemaphoreType.DMA for _ in range(NBUF)],
            [pltpu.SemaphoreType.DMA for _ in range(NBUF)],
        ],
        compiler_params=pltpu.CompilerParams(
            use_tc_tiling_on_sc=False, needs_layout_passes=False
        ),
    )
    def k(idxt_hbm, tok_hbm, pos_hbm, rowc_hbm, out_hbm, idx_v, idxg_v, rows_v,
          out_v, pos_v, rowc_v, gsem, osem):
        wid = lax.axis_index("s") * NC + lax.axis_index("c")
        base = wid * PER_W
        pltpu.sync_copy(pos_hbm, pos_v)
        pltpu.sync_copy(rowc_hbm, rowc_v)
        row_vecs = [
            [rowc_v[j * G + q, pl.ds(0, LANES)] for q in range(G)]
            for j in range(NJ)
        ]

        def start_gather(u, p):
            s = u // (BT // G)
            bt0 = (u % (BT // G)) * G
            pltpu.sync_copy(idxt_hbm.at[s, pl.ds(bt0 * 128, CH)],
                            idx_v[p].at[pl.ds(0, CH)])
            for v in range(CH // LANES):
                t = idx_v[p][pl.ds(v * LANES, LANES)]
                idxg_v[p][pl.ds(v * LANES, LANES)] = jnp.where(
                    t >= HALF, t - HALF, t
                )
            pltpu.async_copy(tok_hbm.at[idxg_v[p]], rows_v[p], gsem[p])

        def work(u, p):
            s = u // (BT // G)
            bt0 = (u % (BT // G)) * G
            pos_j = [pos_v[s, pl.ds(j * LANES, LANES)] for j in range(NJ)]

            def row_body(r, c):
                col = jnp.broadcast_to(r, (LANES,))
                for q in range(G):
                    toko = idx_v[p][pl.ds(q * 128 + r, LANES)][0]
                    off = jnp.where(toko >= HALF, D, 0)
                    for j in range(NJ):
                        vec = rows_v[p][q * 128 + r, pl.ds(off + j * LANES, LANES)]
                        plsc.store_scatter(
                            out_v[p],
                            [row_vecs[j][q], col],
                            vec + pos_j[j],
                        )
                return c

            lax.fori_loop(0, 128, row_body, 0, unroll=4)
            copies = [
                pltpu.async_copy(
                    out_v[p].at[pl.ds(q * D + dt * 8, 8), pl.ds(0, 128)],
                    out_hbm.at[s, pl.ds(dt * (BT * 8) + (bt0 + q) * 8, 8)],
                    osem[p],
                )
                for dt in range(D // 8)
                for q in range(G)
            ]
            for cp in copies:
                cp.wait()

        start_gather(base, 0)

        def pipe(k2, c):
            for p in range(NBUF):
                u = base + k2 + p
                pltpu.make_async_copy(tok_hbm.at[idxg_v[p]], rows_v[p],
                                      gsem[p]).wait()

                @pl.when(u + 1 < base + PER_W)
                def _():
                    start_gather(u + 1, (p + 1) % NBUF)

                work(u, p)
            return c

        lax.fori_loop(0, PER_W // NBUF, lambda i, c: pipe(i * NBUF, c), 0,
                      unroll=False)

    return k


def _rowc() -> np.ndarray:
    t = np.arange(LANES, dtype=np.int32)
    rows = np.empty((NJ * G, LANES), dtype=np.int32)
    for j in range(NJ):
        d = j * LANES + t
        for q in range(G):
            rows[j * G + q] = q * D + d
    return rows


def kernel(inputs, token_table, position_table):
    idxt = jnp.transpose(inputs).astype(jnp.int32)
    tok_t = jnp.transpose(token_table)
    tok_pair = _tc_linearize()(tok_t, tok_t)
    out3 = _build()(idxt, tok_pair, position_table, jnp.asarray(_rowc()))
    out5 = jnp.reshape(out3, (SEQ, D // 8, BT, 8, 128))
    out = jnp.transpose(out5, (2, 4, 0, 1, 3))
    return jnp.reshape(out, (BATCH, SEQ, D))


# speedup vs baseline: 1.3632x; 1.3632x over previous
import functools

import jax
import jax.numpy as jnp
import numpy as np
from jax import lax
from jax.experimental import pallas as pl
from jax.experimental.pallas import tpu as pltpu
from jax.experimental.pallas import tpu_sc as plsc

SEQ = 200
D = 64
BATCH = 4096
LANES = 16
NC, NS = 2, 16
NW = NC * NS

G = 2
CH = G * 128
BT = BATCH // 128
UNITS = SEQ * (BT // G)
PER_W = UNITS // NW
NBUF = 2
NJ = D // LANES


def _build():
    mesh = plsc.VectorSubcoreMesh(core_axis_name="c", subcore_axis_name="s")

    @functools.partial(
        pl.kernel,
        out_type=jax.ShapeDtypeStruct((SEQ, D // 8 * BT * 8, 128), jnp.float32),
        mesh=mesh,
        scratch_types=[
            [pltpu.VMEM((CH,), jnp.int32) for _ in range(NBUF)],
            [pltpu.VMEM((CH, D), jnp.float32) for _ in range(NBUF)],
            [pltpu.VMEM((G * D, 129), jnp.float32) for _ in range(NBUF)],
            pltpu.VMEM((SEQ, D), jnp.float32),
            pltpu.VMEM((NJ * G, LANES), jnp.int32),
            [pltpu.SemaphoreType.DMA for _ in range(NBUF)],
            [pltpu.SemaphoreType.DMA for _ in range(NBUF)],
        ],
        compiler_params=pltpu.CompilerParams(
            use_tc_tiling_on_sc=False, needs_layout_passes=False
        ),
    )
    def k(idxt_hbm, tok_hbm, pos_hbm, rowc_hbm, out_hbm, idx_v, rows_v, out_v,
          pos_v, rowc_v, gsem, osem):
        wid = lax.axis_index("s") * NC + lax.axis_index("c")
        base = wid * PER_W
        pltpu.sync_copy(pos_hbm, pos_v)
        pltpu.sync_copy(rowc_hbm, rowc_v)
        row_vecs = [
            [rowc_v[j * G + q, pl.ds(0, LANES)] for q in range(G)]
            for j in range(NJ)
        ]

        def start_gather(u, p):
            s = u // (BT // G)
            bt0 = (u % (BT // G)) * G
            pltpu.sync_copy(idxt_hbm.at[s, pl.ds(bt0 * 128, CH)], idx_v[p])
            pltpu.async_copy(tok_hbm.at[idx_v[p]], rows_v[p], gsem[p])

        def drain_out(u, p):
            for dt in range(D // 8):
                for q in range(G):
                    pltpu.make_async_copy(
                        out_v[p].at[pl.ds(q * D + dt * 8, 8), pl.ds(0, 128)],
                        out_hbm.at[0, pl.ds(dt * (BT * 8) + q * 8, 8)],
                        osem[p],
                    ).wait()

        def work(u, p):
            s = u // (BT // G)
            bt0 = (u % (BT // G)) * G
            pos_j = [pos_v[s, pl.ds(j * LANES, LANES)] for j in range(NJ)]

            def row_body(r, c):
                col = jnp.broadcast_to(r, (LANES,))
                for q in range(G):
                    for j in range(NJ):
                        vec = rows_v[p][q * 128 + r, pl.ds(j * LANES, LANES)]
                        plsc.store_scatter(
                            out_v[p],
                            [row_vecs[j][q], col],
                            vec + pos_j[j],
                        )
                return c

            lax.fori_loop(0, 128, row_body, 0, unroll=8)
            for dt in range(D // 8):
                for q in range(G):
                    pltpu.async_copy(
                        out_v[p].at[pl.ds(q * D + dt * 8, 8), pl.ds(0, 128)],
                        out_hbm.at[s, pl.ds(dt * (BT * 8) + (bt0 + q) * 8, 8)],
                        osem[p],
                    )

        start_gather(base, 0)

        def pipe(k2, c):
            for p in range(NBUF):
                u = base + k2 + p
                pltpu.make_async_copy(tok_hbm.at[idx_v[p]], rows_v[p],
                                      gsem[p]).wait()

                @pl.when(u + 1 < base + PER_W)
                def _():
                    start_gather(u + 1, (p + 1) % NBUF)

                @pl.when(u >= base + NBUF)
                def _():
                    drain_out(u, p)

                work(u, p)
            return c

        lax.fori_loop(0, PER_W // NBUF, lambda i, c: pipe(i * NBUF, c), 0,
                      unroll=False)
        for p in range(NBUF):
            drain_out(0, p)

    return k


def _rowc() -> np.ndarray:
    t = np.arange(LANES, dtype=np.int32)
    rows = np.empty((NJ * G, LANES), dtype=np.int32)
    for j in range(NJ):
        d = j * LANES + t
        for q in range(G):
            rows[j * G + q] = q * D + d
    return rows


def kernel(inputs, token_table, position_table):
    idxt = jnp.transpose(inputs).astype(jnp.int32)
    out3 = _build()(idxt, token_table, position_table, jnp.asarray(_rowc()))
    out5 = jnp.reshape(out3, (SEQ, D // 8, BT, 8, 128))
    out = jnp.transpose(out5, (2, 4, 0, 1, 3))
    return jnp.reshape(out, (BATCH, SEQ, D))


# speedup vs baseline: 1.3679x; 1.0035x over previous
import functools

import jax
import jax.numpy as jnp
import numpy as np
from jax import lax
from jax.experimental import pallas as pl
from jax.experimental.pallas import tpu as pltpu
from jax.experimental.pallas import tpu_sc as plsc

SEQ = 200
D = 64
BATCH = 4096
LANES = 16
NC, NS = 2, 16
NW = NC * NS

G = 2
CH = G * 128
BT = BATCH // 128
UNITS = SEQ * (BT // G)
PER_W = UNITS // NW
NBUF = 2
NJ = D // LANES


def _build():
    mesh = plsc.VectorSubcoreMesh(core_axis_name="c", subcore_axis_name="s")

    @functools.partial(
        pl.kernel,
        out_type=jax.ShapeDtypeStruct((SEQ, D // 8 * BT * 8, 128), jnp.float32),
        mesh=mesh,
        scratch_types=[
            [pltpu.VMEM((CH,), jnp.int32) for _ in range(NBUF)],
            [pltpu.VMEM((CH, D), jnp.float32) for _ in range(NBUF)],
            [pltpu.VMEM((G * D, 129), jnp.float32) for _ in range(NBUF)],
            pltpu.VMEM((SEQ, D), jnp.float32),
            pltpu.VMEM((NJ * G, LANES), jnp.int32),
            [pltpu.SemaphoreType.DMA for _ in range(NBUF)],
            [pltpu.SemaphoreType.DMA for _ in range(NBUF)],
        ],
        compiler_params=pltpu.CompilerParams(
            use_tc_tiling_on_sc=False, needs_layout_passes=False
        ),
    )
    def k(idxt_hbm, tok_hbm, pos_hbm, rowc_hbm, out_hbm, idx_v, rows_v, out_v,
          pos_v, rowc_v, gsem, osem):
        wid = lax.axis_index("s") * NC + lax.axis_index("c")
        base = wid * PER_W
        pltpu.sync_copy(pos_hbm, pos_v)
        pltpu.sync_copy(rowc_hbm, rowc_v)
        row_vecs = [
            [rowc_v[j * G + q, pl.ds(0, LANES)] for q in range(G)]
            for j in range(NJ)
        ]

        def start_gather(u, p):
            s = u // (BT // G)
            bt0 = (u % (BT // G)) * G
            pltpu.sync_copy(idxt_hbm.at[s, pl.ds(bt0 * 128, CH)], idx_v[p])
            pltpu.async_copy(tok_hbm.at[idx_v[p]], rows_v[p], gsem[p])

        def drain_out(u, p):
            for dt in range(D // 8):
                for q in range(G):
                    pltpu.make_async_copy(
                        out_v[p].at[pl.ds(q * D + dt * 8, 8), pl.ds(0, 128)],
                        out_hbm.at[0, pl.ds(dt * (BT * 8) + q * 8, 8)],
                        osem[p],
                    ).wait()

        def work(u, p):
            s = u // (BT // G)
            bt0 = (u % (BT // G)) * G
            pos_j = [pos_v[s, pl.ds(j * LANES, LANES)] for j in range(NJ)]

            def row_body(r, c):
                col = jnp.broadcast_to(r, (LANES,))
                for q in range(G):
                    for j in range(NJ):
                        vec = rows_v[p][q * 128 + r, pl.ds(j * LANES, LANES)]
                        plsc.store_scatter(
                            out_v[p],
                            [row_vecs[j][q], col],
                            vec + pos_j[j],
                        )
                return c

            lax.fori_loop(0, 128, row_body, 0, unroll=16)
            for dt in range(D // 8):
                for q in range(G):
                    pltpu.async_copy(
                        out_v[p].at[pl.ds(q * D + dt * 8, 8), pl.ds(0, 128)],
                        out_hbm.at[s, pl.ds(dt * (BT * 8) + (bt0 + q) * 8, 8)],
                        osem[p],
                    )

        start_gather(base, 0)

        def pipe(k2, c):
            for p in range(NBUF):
                u = base + k2 + p
                pltpu.make_async_copy(tok_hbm.at[idx_v[p]], rows_v[p],
                                      gsem[p]).wait()

                @pl.when(u + 1 < base + PER_W)
                def _():
                    start_gather(u + 1, (p + 1) % NBUF)

                @pl.when(u >= base + NBUF)
                def _():
                    drain_out(u, p)

                work(u, p)
            return c

        lax.fori_loop(0, PER_W // NBUF, lambda i, c: pipe(i * NBUF, c), 0,
                      unroll=False)
        for p in range(NBUF):
            drain_out(0, p)

    return k


def _rowc() -> np.ndarray:
    t = np.arange(LANES, dtype=np.int32)
    rows = np.empty((NJ * G, LANES), dtype=np.int32)
    for j in range(NJ):
        d = j * LANES + t
        for q in range(G):
            rows[j * G + q] = q * D + d
    return rows


def kernel(inputs, token_table, position_table):
    idxt = jnp.transpose(inputs).astype(jnp.int32)
    out3 = _build()(idxt, token_table, position_table, jnp.asarray(_rowc()))
    out5 = jnp.reshape(out3, (SEQ, D // 8, BT, 8, 128))
    out = jnp.transpose(out5, (2, 4, 0, 1, 3))
    return jnp.reshape(out, (BATCH, SEQ, D))


# speedup vs baseline: 1.9491x; 1.4248x over previous
import functools

import jax
import jax.numpy as jnp
import numpy as np
from jax import lax
from jax.experimental import pallas as pl
from jax.experimental.pallas import tpu as pltpu
from jax.experimental.pallas import tpu_sc as plsc

SEQ = 200
D = 64
BATCH = 4096
LANES = 16
NC, NS = 2, 16
NW = NC * NS

G = 2
CH = G * 128
BT = BATCH // 128
UNITS = SEQ * (BT // G)
PER_W = UNITS // NW
NBUF = 2
NJ = D // LANES


def _build():
    mesh = plsc.VectorSubcoreMesh(core_axis_name="c", subcore_axis_name="s")

    @functools.partial(
        pl.kernel,
        out_type=jax.ShapeDtypeStruct((SEQ, D // 8 * BT * 8, 128), jnp.float32),
        mesh=mesh,
        scratch_types=[
            [pltpu.VMEM((CH,), jnp.int32) for _ in range(NBUF)],
            [pltpu.VMEM((CH, D), jnp.float32) for _ in range(NBUF)],
            [pltpu.VMEM((G * D, 129), jnp.float32) for _ in range(NBUF)],
            pltpu.VMEM((SEQ, D), jnp.float32),
            pltpu.VMEM((NJ * G, LANES), jnp.int32),
            [pltpu.SemaphoreType.DMA for _ in range(NBUF)],
            [pltpu.SemaphoreType.DMA for _ in range(NBUF)],
        ],
        compiler_params=pltpu.CompilerParams(
            use_tc_tiling_on_sc=False, needs_layout_passes=False
        ),
    )
    def k(idxt_hbm, tok_hbm, pos_hbm, rowc_hbm, out_hbm, idx_v, rows_v, out_v,
          pos_v, rowc_v, gsem, osem):
        wid = lax.axis_index("s") * NC + lax.axis_index("c")
        base = wid * PER_W
        pltpu.sync_copy(pos_hbm, pos_v)
        pltpu.sync_copy(rowc_hbm, rowc_v)
        row_vecs = [
            [rowc_v[j * G + q, pl.ds(0, LANES)] for q in range(G)]
            for j in range(NJ)
        ]

        def start_gather(u, p):
            s = u // (BT // G)
            bt0 = (u % (BT // G)) * G
            pltpu.sync_copy(idxt_hbm.at[s, pl.ds(bt0 * 128, CH)], idx_v[p])
            pltpu.async_copy(tok_hbm.at[idx_v[p]], rows_v[p], gsem[p])

        def drain_out(u, p):
            for dt in range(D // 8):
                for q in range(G):
                    pltpu.make_async_copy(
                        out_v[p].at[pl.ds(q * D + dt * 8, 8), pl.ds(0, 128)],
                        out_hbm.at[0, pl.ds(dt * (BT * 8) + q * 8, 8)],
                        osem[p],
                    ).wait()

        def work(u, p):
            s = u // (BT // G)
            bt0 = (u % (BT // G)) * G
            pos_j = [pos_v[s, pl.ds(j * LANES, LANES)] for j in range(NJ)]

            @plsc.parallel_loop(0, 128, unroll=8)
            def _(r):
                col = jnp.broadcast_to(r, (LANES,))
                for q in range(G):
                    for j in range(NJ):
                        vec = rows_v[p][q * 128 + r, pl.ds(j * LANES, LANES)]
                        plsc.store_scatter(
                            out_v[p],
                            [row_vecs[j][q], col],
                            vec + pos_j[j],
                        )
            for dt in range(D // 8):
                for q in range(G):
                    pltpu.async_copy(
                        out_v[p].at[pl.ds(q * D + dt * 8, 8), pl.ds(0, 128)],
                        out_hbm.at[s, pl.ds(dt * (BT * 8) + (bt0 + q) * 8, 8)],
                        osem[p],
                    )

        start_gather(base, 0)

        def pipe(k2, c):
            for p in range(NBUF):
                u = base + k2 + p
                pltpu.make_async_copy(tok_hbm.at[idx_v[p]], rows_v[p],
                                      gsem[p]).wait()

                @pl.when(u + 1 < base + PER_W)
                def _():
                    start_gather(u + 1, (p + 1) % NBUF)

                @pl.when(u >= base + NBUF)
                def _():
                    drain_out(u, p)

                work(u, p)
            return c

        lax.fori_loop(0, PER_W // NBUF, lambda i, c: pipe(i * NBUF, c), 0,
                      unroll=False)
        for p in range(NBUF):
            drain_out(0, p)

    return k


def _rowc() -> np.ndarray:
    t = np.arange(LANES, dtype=np.int32)
    rows = np.empty((NJ * G, LANES), dtype=np.int32)
    for j in range(NJ):
        d = j * LANES + t
        for q in range(G):
            rows[j * G + q] = q * D + d
    return rows


def kernel(inputs, token_table, position_table):
    idxt = jnp.transpose(inputs).astype(jnp.int32)
    out3 = _build()(idxt, token_table, position_table, jnp.asarray(_rowc()))
    out5 = jnp.reshape(out3, (SEQ, D // 8, BT, 8, 128))
    out = jnp.transpose(out5, (2, 4, 0, 1, 3))
    return jnp.reshape(out, (BATCH, SEQ, D))
